# baseline (speedup 1.0000x reference)
import functools

import jax
import jax.numpy as jnp
from jax import lax
from jax.experimental import pallas as pl
from jax.experimental.pallas import tpu as pltpu
from jax.experimental.pallas import tpu_sc as plsc

BATCH = 16384
HIST = 50
EMBED_DIM = 32
TOTAL = BATCH * HIST
NUM_CORES = 2
NUM_SUBCORES = 16
NW = NUM_CORES * NUM_SUBCORES
B_PER_W = BATCH // NW
NCHUNK16 = B_PER_W // 16


def _emb_body(idx_hbm, table_hbm, out_hbm, idx_all, rows_v, t_v, isem, gsem):
    wid = lax.axis_index("s") * NUM_CORES + lax.axis_index("c")
    b0 = wid * B_PER_W
    icps = [
        pltpu.async_copy(
            idx_hbm.at[pl.ds(h * BATCH + b0, B_PER_W)], idx_all.at[h], isem
        )
        for h in range(HIST)
    ]
    for cp in icps:
        cp.wait()
    iota16 = lax.iota(jnp.int32, 16)

    def h_step(h, carry):
        pltpu.async_copy(table_hbm.at[idx_all.at[h]], rows_v, gsem).wait()

        def t_step(bc, carry2):
            b_ids = bc * 16 + iota16
            for c in range(EMBED_DIM):
                c_ids = jnp.full((16,), c, jnp.int32)
                vals = plsc.load_gather(rows_v, [b_ids, c_ids])
                t_v.at[c][pl.ds(bc * 16, 16)] = vals
            return carry2

        lax.fori_loop(0, NCHUNK16, t_step, 0)
        pltpu.sync_copy(t_v, out_hbm.at[h, :, pl.ds(b0, B_PER_W)])
        return carry

    lax.fori_loop(0, HIST, h_step, 0)


@jax.jit
def kernel(input, weight):
    idx = input.T.reshape(TOTAL)
    mesh = plsc.VectorSubcoreMesh(core_axis_name="c", subcore_axis_name="s")
    run = pl.kernel(
        _emb_body,
        out_type=jax.ShapeDtypeStruct((HIST, EMBED_DIM, BATCH), jnp.float32),
        mesh=mesh,
        scratch_types=[
            pltpu.VMEM((HIST, B_PER_W), jnp.int32),
            pltpu.VMEM((B_PER_W, EMBED_DIM), jnp.float32),
            pltpu.VMEM((EMBED_DIM, B_PER_W), jnp.float32),
            pltpu.SemaphoreType.DMA,
            pltpu.SemaphoreType.DMA,
        ],
        compiler_params=pltpu.CompilerParams(
            use_tc_tiling_on_sc=False, needs_layout_passes=False
        ),
    )
    out = run(idx, weight)
    return out.transpose(2, 0, 1)


# speedup vs baseline: 1.0553x; 1.0553x over previous
import functools

import jax
import jax.numpy as jnp
from jax import lax
from jax.experimental import pallas as pl
from jax.experimental.pallas import tpu as pltpu
from jax.experimental.pallas import tpu_sc as plsc

BATCH = 16384
HIST = 50
EMBED_DIM = 32
TOTAL = BATCH * HIST
NUM_CORES = 2
NUM_SUBCORES = 16
NW = NUM_CORES * NUM_SUBCORES
B_PER_W = BATCH // NW
NCHUNK16 = B_PER_W // 16


def _emb_body(
    idx_hbm, table_hbm, out_hbm, idx_all, rows_v0, rows_v1, t_v, isem, gsem0, gsem1
):
    wid = lax.axis_index("s") * NUM_CORES + lax.axis_index("c")
    b0 = wid * B_PER_W
    icps = [
        pltpu.async_copy(
            idx_hbm.at[pl.ds(h * BATCH + b0, B_PER_W)], idx_all.at[h], isem
        )
        for h in range(HIST)
    ]
    for cp in icps:
        cp.wait()
    iota16 = lax.iota(jnp.int32, 16)

    def transpose_and_store(h, rows_v):
        def t_step(bc, carry2):
            b_ids = bc * 16 + iota16
            for c in range(EMBED_DIM):
                c_ids = jnp.full((16,), c, jnp.int32)
                vals = plsc.load_gather(rows_v, [b_ids, c_ids])
                t_v.at[c][pl.ds(bc * 16, 16)] = vals
            return carry2

        lax.fori_loop(0, NCHUNK16, t_step, 0)
        pltpu.sync_copy(t_v, out_hbm.at[h, :, pl.ds(b0, B_PER_W)])

    pltpu.async_copy(table_hbm.at[idx_all.at[0]], rows_v0, gsem0).wait()

    def h_step(i, carry):
        h0 = 2 * i
        g1 = pltpu.async_copy(
            table_hbm.at[idx_all.at[h0 + 1]], rows_v1, gsem1
        )
        transpose_and_store(h0, rows_v0)
        g0 = pltpu.async_copy(
            table_hbm.at[idx_all.at[jnp.minimum(h0 + 2, HIST - 1)]], rows_v0, gsem0
        )
        g1.wait()
        transpose_and_store(h0 + 1, rows_v1)
        g0.wait()
        return carry

    lax.fori_loop(0, HIST // 2, h_step, 0)


@jax.jit
def kernel(input, weight):
    idx = input.T.reshape(TOTAL)
    mesh = plsc.VectorSubcoreMesh(core_axis_name="c", subcore_axis_name="s")
    run = pl.kernel(
        _emb_body,
        out_type=jax.ShapeDtypeStruct((HIST, EMBED_DIM, BATCH), jnp.float32),
        mesh=mesh,
        scratch_types=[
            pltpu.VMEM((HIST, B_PER_W), jnp.int32),
            pltpu.VMEM((B_PER_W, EMBED_DIM), jnp.float32),
            pltpu.VMEM((B_PER_W, EMBED_DIM), jnp.float32),
            pltpu.VMEM((EMBED_DIM, B_PER_W), jnp.float32),
            pltpu.SemaphoreType.DMA,
            pltpu.SemaphoreType.DMA,
            pltpu.SemaphoreType.DMA,
        ],
        compiler_params=pltpu.CompilerParams(
            use_tc_tiling_on_sc=False, needs_layout_passes=False
        ),
    )
    out = run(idx, weight)
    return out.transpose(2, 0, 1)


# speedup vs baseline: 1.2716x; 1.2050x over previous
import functools

import jax
import jax.numpy as jnp
from jax import lax
from jax.experimental import pallas as pl
from jax.experimental.pallas import tpu as pltpu
from jax.experimental.pallas import tpu_sc as plsc

BATCH = 16384
HIST = 50
HIST_PAD = 56
EMBED_DIM = 32
TOTAL = BATCH * HIST_PAD
NUM_CORES = 2
NUM_SUBCORES = 16
NW = NUM_CORES * NUM_SUBCORES
B_PER_W = BATCH // NW
NB = 32
NSTEP = B_PER_W // NB
CHUNK = NB * HIST_PAD


def _emb_body(idx_hbm, table_hbm, out_hbm, idx_v, rows_v, sem, osem):
    wid = lax.axis_index("s") * NUM_CORES + lax.axis_index("c")
    base = wid * B_PER_W
    for j in range(NSTEP):
        b0 = base + j * NB
        pltpu.sync_copy(idx_hbm.at[pl.ds(b0 * HIST_PAD, CHUNK)], idx_v)
        pltpu.async_copy(table_hbm.at[idx_v], rows_v, sem).wait()
        cps = [
            pltpu.async_copy(
                rows_v.at[pl.ds(b * HIST_PAD, HIST_PAD), :],
                out_hbm.at[b0 + b],
                osem,
            )
            for b in range(NB)
        ]
        for cp in cps:
            cp.wait()


@jax.jit
def kernel(input, weight):
    npad = HIST_PAD - HIST
    pad = jnp.arange(BATCH * npad, dtype=jnp.int32).reshape(BATCH, npad)
    idx = jnp.concatenate([input, pad], axis=1).reshape(TOTAL)
    mesh = plsc.VectorSubcoreMesh(core_axis_name="c", subcore_axis_name="s")
    run = pl.kernel(
        _emb_body,
        out_type=jax.ShapeDtypeStruct((BATCH, HIST_PAD, EMBED_DIM), jnp.float32),
        mesh=mesh,
        scratch_types=[
            pltpu.VMEM((CHUNK,), jnp.int32),
            pltpu.VMEM((CHUNK, EMBED_DIM), jnp.float32),
            pltpu.SemaphoreType.DMA,
            pltpu.SemaphoreType.DMA,
        ],
        compiler_params=pltpu.CompilerParams(use_tc_tiling_on_sc=False),
    )
    out = run(idx, weight)
    return out[:, :HIST, :]
